# baseline (speedup 1.0000x reference)
import jax
import jax.numpy as jnp
from jax.experimental import pallas as pl
from jax.experimental.pallas import tpu as pltpu

E = 64
K = 8
NBUF = 4
CH = 512


def _router(x_hbm, w_ref, b_ref, scores_ref, idx_ref, bufs, sems):
    nchunk = x_hbm.shape[0] // CH

    def start(c, slot):
        pltpu.make_async_copy(
            x_hbm.at[pl.ds(c * CH, CH), :], bufs.at[slot], sems.at[slot]
        ).start()

    def wait(slot):
        pltpu.make_async_copy(
            x_hbm.at[pl.ds(0, CH), :], bufs.at[slot], sems.at[slot]
        ).wait()

    w = w_ref[...]
    b = b_ref[...]

    for s in range(min(NBUF, nchunk)):
        start(s, s)

    for c in range(nchunk):
        slot = c % NBUF
        wait(slot)
        x = bufs[slot]
        lt = jax.lax.dot_general(
            w, x, (((1,), (1,)), ((), ())),
            preferred_element_type=jnp.float32)
        lt = lt + b[:, None]
        if c + NBUF < nchunk:
            start(c + NBUF, slot)

        m = jnp.max(lt, axis=0, keepdims=True)
        ex = jnp.exp(lt - m)
        scores_t = ex / jnp.sum(ex, axis=0, keepdims=True)
        scores_ref[pl.ds(c * CH, CH), :] = scores_t.T

        iota = jax.lax.broadcasted_iota(jnp.int32, (E, CH), 0)
        s = scores_t
        neg = jnp.float32(-jnp.inf)
        for k in range(K):
            mk = jnp.max(s, axis=0, keepdims=True)
            cand = jnp.where(s == mk, iota, E)
            amin = jnp.min(cand, axis=0, keepdims=True)
            idx_ref[k, pl.ds(c * CH, CH)] = amin[0]
            s = jnp.where(iota == amin, neg, s)


def kernel(hidden_states, weight, bias):
    Bn, Sn, Hn = hidden_states.shape
    T = Bn * Sn
    flat = hidden_states.reshape(T, Hn)

    scores, idx_t = pl.pallas_call(
        _router,
        grid=(),
        in_specs=[
            pl.BlockSpec(memory_space=pltpu.MemorySpace.HBM),
            pl.BlockSpec(memory_space=pltpu.MemorySpace.VMEM),
            pl.BlockSpec(memory_space=pltpu.MemorySpace.VMEM),
        ],
        out_specs=[
            pl.BlockSpec(memory_space=pltpu.MemorySpace.VMEM),
            pl.BlockSpec(memory_space=pltpu.MemorySpace.VMEM),
        ],
        out_shape=[
            jax.ShapeDtypeStruct((T, E), jnp.float32),
            jax.ShapeDtypeStruct((K, T), jnp.int32),
        ],
        scratch_shapes=[
            pltpu.VMEM((NBUF, CH, Hn), jnp.float32),
            pltpu.SemaphoreType.DMA((NBUF,)),
        ],
    )(flat, weight, bias)
    return (scores, idx_t.T)


# speedup vs baseline: 1.0003x; 1.0003x over previous
import jax
import jax.numpy as jnp
from jax.experimental import pallas as pl
from jax.experimental.pallas import tpu as pltpu

E = 64
K = 8
NBUF = 6
CH = 512


def _router(x_hbm, w_ref, b_ref, scores_ref, idx_ref, bufs, sems):
    nchunk = x_hbm.shape[0] // CH

    def start(c, slot):
        pltpu.make_async_copy(
            x_hbm.at[pl.ds(c * CH, CH), :], bufs.at[slot], sems.at[slot]
        ).start()

    def wait(slot):
        pltpu.make_async_copy(
            x_hbm.at[pl.ds(0, CH), :], bufs.at[slot], sems.at[slot]
        ).wait()

    w = w_ref[...]
    b = b_ref[...]

    for s in range(min(NBUF, nchunk)):
        start(s, s)

    for c in range(nchunk):
        slot = c % NBUF
        wait(slot)
        x = bufs[slot]
        lt = jax.lax.dot_general(
            w, x, (((1,), (1,)), ((), ())),
            preferred_element_type=jnp.float32)
        lt = lt + b[:, None]
        if c + NBUF < nchunk:
            start(c + NBUF, slot)

        m = jnp.max(lt, axis=0, keepdims=True)
        ex = jnp.exp(lt - m)
        scores_t = ex / jnp.sum(ex, axis=0, keepdims=True)
        scores_ref[pl.ds(c * CH, CH), :] = scores_t.T

        iota = jax.lax.broadcasted_iota(jnp.int32, (E, CH), 0)
        s = scores_t
        neg = jnp.float32(-jnp.inf)
        for k in range(K):
            mk = jnp.max(s, axis=0, keepdims=True)
            cand = jnp.where(s == mk, iota, E)
            amin = jnp.min(cand, axis=0, keepdims=True)
            idx_ref[k, pl.ds(c * CH, CH)] = amin[0]
            s = jnp.where(iota == amin, neg, s)


def kernel(hidden_states, weight, bias):
    Bn, Sn, Hn = hidden_states.shape
    T = Bn * Sn
    flat = hidden_states.reshape(T, Hn)

    scores, idx_t = pl.pallas_call(
        _router,
        grid=(),
        in_specs=[
            pl.BlockSpec(memory_space=pltpu.MemorySpace.HBM),
            pl.BlockSpec(memory_space=pltpu.MemorySpace.VMEM),
            pl.BlockSpec(memory_space=pltpu.MemorySpace.VMEM),
        ],
        out_specs=[
            pl.BlockSpec(memory_space=pltpu.MemorySpace.VMEM),
            pl.BlockSpec(memory_space=pltpu.MemorySpace.VMEM),
        ],
        out_shape=[
            jax.ShapeDtypeStruct((T, E), jnp.float32),
            jax.ShapeDtypeStruct((K, T), jnp.int32),
        ],
        scratch_shapes=[
            pltpu.VMEM((NBUF, CH, Hn), jnp.float32),
            pltpu.SemaphoreType.DMA((NBUF,)),
        ],
    )(flat, weight, bias)
    return (scores, idx_t.T)
